# baseline (speedup 1.0000x reference)
import functools

import jax
import jax.numpy as jnp
from jax import lax
from jax.experimental import pallas as pl
from jax.experimental.pallas import tpu as pltpu
from jax.experimental.pallas import tpu_sc as plsc

NC = 2
NS = 16
NW = NC * NS
LANES = 16
CHUNK = 128


@functools.lru_cache(maxsize=None)
def _build_glove_sc(B: int, D: int):
    b_per_w = B // NW
    n_chunks = b_per_w // CHUNK
    n_seg = D // LANES
    mesh = plsc.VectorSubcoreMesh(
        core_axis_name="c", subcore_axis_name="s",
        num_cores=NC, num_subcores=NS,
    )

    @functools.partial(
        pl.kernel,
        out_type=jax.ShapeDtypeStruct((B,), jnp.float32),
        mesh=mesh,
        compiler_params=pltpu.CompilerParams(
            needs_layout_passes=False, use_tc_tiling_on_sc=False),
        scratch_types=[
            pltpu.VMEM((n_chunks, CHUNK), jnp.int32),
            pltpu.VMEM((n_chunks, CHUNK), jnp.int32),
            pltpu.VMEM((b_per_w, D), jnp.float32),
            pltpu.VMEM((b_per_w, D), jnp.float32),
            pltpu.VMEM((b_per_w,), jnp.float32),
            pltpu.VMEM((b_per_w,), jnp.float32),
            pltpu.VMEM((b_per_w,), jnp.float32),
            pltpu.SemaphoreType.DMA,
        ],
    )
    def glove_kernel(tok_hbm, ctx_hbm, temb_hbm, cemb_hbm, tb_hbm, cb_hbm,
                     out_hbm, idx_i, idx_j, wi_v, wj_v, bi_v, bj_v,
                     out_v, sem):
        wid = lax.axis_index("s") * NC + lax.axis_index("c")

        pltpu.sync_copy(tok_hbm.at[wid], idx_i)
        pltpu.sync_copy(ctx_hbm.at[wid], idx_j)

        copies = []
        for c in range(n_chunks):
            dst = pl.ds(c * CHUNK, CHUNK)
            copies.append(
                pltpu.async_copy(temb_hbm.at[idx_i.at[c]], wi_v.at[dst], sem))
            copies.append(
                pltpu.async_copy(cemb_hbm.at[idx_j.at[c]], wj_v.at[dst], sem))
            copies.append(
                pltpu.async_copy(tb_hbm.at[idx_i.at[c]], bi_v.at[dst], sem))
            copies.append(
                pltpu.async_copy(cb_hbm.at[idx_j.at[c]], bj_v.at[dst], sem))
        for cp in copies:
            cp.wait()

        lane_ids = lax.iota(jnp.int32, LANES)

        def block(b, carry):
            r0 = b * LANES
            sums = jnp.zeros((LANES,), jnp.float32)
            for r in range(LANES):
                row = r0 + r
                acc = wi_v[row, pl.ds(0, LANES)] * wj_v[row, pl.ds(0, LANES)]
                for s in range(1, n_seg):
                    sl = pl.ds(s * LANES, LANES)
                    acc = acc + wi_v[row, sl] * wj_v[row, sl]
                sums = jnp.where(lane_ids == r, jnp.sum(acc), sums)
            blk = pl.ds(r0, LANES)
            out_v[blk] = sums + bi_v[blk] + bj_v[blk]
            return carry

        lax.fori_loop(0, b_per_w // LANES, block, 0)
        pltpu.sync_copy(out_v, out_hbm.at[pl.ds(wid * b_per_w, b_per_w)])

    return glove_kernel


def kernel(token, context_token, token_embedding, context_embedding,
           token_bias, context_bias):
    B = token.shape[0]
    D = token_embedding.shape[1]
    tok = token.astype(jnp.int32).reshape(NW, -1, CHUNK)
    ctx = context_token.astype(jnp.int32).reshape(NW, -1, CHUNK)
    tb = token_bias.reshape(-1)
    cb = context_bias.reshape(-1)
    return _build_glove_sc(B, D)(tok, ctx, token_embedding,
                                 context_embedding, tb, cb)
